# baseline (speedup 1.0000x reference)
import functools

import jax
import jax.numpy as jnp
from jax import lax
from jax.experimental import pallas as pl
from jax.experimental.pallas import tpu as pltpu
from jax.experimental.pallas import tpu_sc as plsc

_NUM_DOMAINS = 1000
_NUM_STYLES = 100000
_DOM_DIM = 64
_STYLE_DIM = 128
_COND_DIM = 1024
_BATCH = 16384

_CHUNK = 128


def _make_gather(batch):
    info = plsc.get_sparse_core_info()
    nw = info.num_cores * info.num_subcores
    b_per_w = batch // nw
    nch = b_per_w // _CHUNK

    mesh = plsc.VectorSubcoreMesh(core_axis_name="c", subcore_axis_name="s")

    @functools.partial(
        pl.kernel,
        mesh=mesh,
        out_type=[
            jax.ShapeDtypeStruct((batch, _STYLE_DIM), jnp.float32),
            jax.ShapeDtypeStruct((batch, _STYLE_DIM), jnp.float32),
        ],
        scratch_types=[
            pltpu.VMEM((nch, _CHUNK), jnp.int32),
            pltpu.VMEM((nch, _CHUNK), jnp.int32),
            pltpu.VMEM((2, _CHUNK, _STYLE_DIM), jnp.float32),
            pltpu.VMEM((2, _CHUNK, _STYLE_DIM), jnp.float32),
            pltpu.VMEM_SHARED((_NUM_DOMAINS, _STYLE_DIM), jnp.float32),
            pltpu.SemaphoreType.DMA,
            pltpu.SemaphoreType.DMA,
            pltpu.SemaphoreType.DMA,
            pltpu.SemaphoreType.DMA,
        ],
    )
    def gather(dom_tab, sty_tab, dom_id, sty_id, dom_out, sty_out,
               idx_d, idx_s, dom_v, sty_v, dom_sh, sd0, sd1, ss0, ss1):
        sid = lax.axis_index("s")
        wid = sid * info.num_cores + lax.axis_index("c")
        base = wid * b_per_w
        sem_d = (sd0, sd1)
        sem_s = (ss0, ss1)

        @pl.when(sid == 0)
        def _():
            pltpu.sync_copy(dom_tab, dom_sh)

        pltpu.sync_copy(dom_id.at[wid], idx_d)
        pltpu.sync_copy(sty_id.at[wid], idx_s)
        plsc.subcore_barrier()

        def fire(j, buf):
            cd = pltpu.async_copy(dom_sh.at[idx_d.at[j]], dom_v.at[buf],
                                  sem_d[buf])
            cs = pltpu.async_copy(sty_tab.at[idx_s.at[j]], sty_v.at[buf],
                                  sem_s[buf])
            return cd, cs

        pend = fire(0, 0)
        for j in range(nch):
            cur = pend
            if j + 1 < nch:
                pend = fire(j + 1, (j + 1) % 2)
            cur[0].wait()
            cur[1].wait()
            buf = j % 2
            pltpu.sync_copy(dom_v.at[buf],
                            dom_out.at[pl.ds(base + j * _CHUNK, _CHUNK)])
            pltpu.sync_copy(sty_v.at[buf],
                            sty_out.at[pl.ds(base + j * _CHUNK, _CHUNK)])

    def run(dom_tab, sty_tab, dom_id, sty_id):
        dom_tab_p = jnp.pad(dom_tab, ((0, 0), (0, _STYLE_DIM - _DOM_DIM)))
        dom_id_r = dom_id.astype(jnp.int32).reshape(nw, nch, _CHUNK)
        sty_id_r = sty_id.astype(jnp.int32).reshape(nw, nch, _CHUNK)
        return gather(dom_tab_p, sty_tab, dom_id_r, sty_id_r)

    return run


def _mlp_body(dom_ref, sty_ref, w1a_ref, w1b_ref, b1_ref, w2_ref, b2_ref,
              out_ref):
    h = (jnp.dot(dom_ref[:, :_DOM_DIM], w1a_ref[...],
                 preferred_element_type=jnp.float32)
         + jnp.dot(sty_ref[...], w1b_ref[...],
                   preferred_element_type=jnp.float32)
         + b1_ref[...])
    a = h * 0.5 * (1.0 + jnp.tanh(0.5 * h))
    out_ref[...] = (jnp.dot(a, w2_ref[...],
                            preferred_element_type=jnp.float32)
                    + b2_ref[...])


def _mlp_slice(buf, dom, sty, w1a, w1b, b1, w2, b2, blk_off, total,
               block_b=1024):
    nblk = dom.shape[0] // block_b
    off = blk_off
    data_specs = [
        pl.BlockSpec((block_b, _STYLE_DIM), lambda i: (i, 0)),
        pl.BlockSpec((block_b, _STYLE_DIM), lambda i: (i, 0)),
        pl.BlockSpec((_DOM_DIM, _COND_DIM), lambda i: (0, 0)),
        pl.BlockSpec((_STYLE_DIM, _COND_DIM), lambda i: (0, 0)),
        pl.BlockSpec((1, _COND_DIM), lambda i: (0, 0)),
        pl.BlockSpec((_COND_DIM, _COND_DIM), lambda i: (0, 0)),
        pl.BlockSpec((1, _COND_DIM), lambda i: (0, 0)),
    ]
    out_spec = pl.BlockSpec((block_b, _COND_DIM), lambda i: (i + off, 0))
    out_shape = jax.ShapeDtypeStruct((total, _COND_DIM), jnp.float32)
    params = pltpu.CompilerParams(dimension_semantics=("arbitrary",))
    if buf is None:
        return pl.pallas_call(
            _mlp_body,
            grid=(nblk,),
            in_specs=data_specs,
            out_specs=out_spec,
            out_shape=out_shape,
            compiler_params=params,
        )(dom, sty, w1a, w1b, b1, w2, b2)

    def body(buf_ref, *refs):
        del buf_ref
        _mlp_body(*refs)

    return pl.pallas_call(
        body,
        grid=(nblk,),
        in_specs=[pl.BlockSpec(memory_space=pl.ANY)] + data_specs,
        out_specs=out_spec,
        out_shape=out_shape,
        input_output_aliases={0: 0},
        compiler_params=params,
    )(buf, dom, sty, w1a, w1b, b1, w2, b2)


_HALF = _BATCH // 2
_gather_half = _make_gather(_HALF)


def kernel(domain_id, style_id, domain_table, style_table, W1, b1, W2, b2):
    d1, s1 = _gather_half(domain_table, style_table,
                          domain_id[:_HALF], style_id[:_HALF])
    d2, s2 = _gather_half(domain_table, style_table,
                          domain_id[_HALF:], style_id[_HALF:])
    w1a, w1b = W1[:_DOM_DIM], W1[_DOM_DIM:]
    b1r, b2r = b1.reshape(1, -1), b2.reshape(1, -1)
    nblk_half = _HALF // 1024
    out = _mlp_slice(None, d1, s1, w1a, w1b, b1r, W2, b2r,
                     blk_off=0, total=_BATCH)
    out = _mlp_slice(out, d2, s2, w1a, w1b, b1r, W2, b2r,
                     blk_off=nblk_half, total=_BATCH)
    return out


# speedup vs baseline: 1.0315x; 1.0315x over previous
import functools

import jax
import jax.numpy as jnp
from jax import lax
from jax.experimental import pallas as pl
from jax.experimental.pallas import tpu as pltpu
from jax.experimental.pallas import tpu_sc as plsc

_NUM_DOMAINS = 1000
_NUM_STYLES = 100000
_DOM_DIM = 64
_STYLE_DIM = 128
_COND_DIM = 1024
_BATCH = 16384

_CHUNK = 128


def _make_gather(batch):
    info = plsc.get_sparse_core_info()
    nw = info.num_cores * info.num_subcores
    b_per_w = batch // nw
    nch = b_per_w // _CHUNK

    mesh = plsc.VectorSubcoreMesh(core_axis_name="c", subcore_axis_name="s")

    @functools.partial(
        pl.kernel,
        mesh=mesh,
        out_type=[
            jax.ShapeDtypeStruct((batch, _STYLE_DIM), jnp.float32),
            jax.ShapeDtypeStruct((batch, _STYLE_DIM), jnp.float32),
        ],
        scratch_types=[
            pltpu.VMEM((nch, _CHUNK), jnp.int32),
            pltpu.VMEM((nch, _CHUNK), jnp.int32),
            pltpu.VMEM((2, _CHUNK, _STYLE_DIM), jnp.float32),
            pltpu.VMEM((2, _CHUNK, _STYLE_DIM), jnp.float32),
            pltpu.VMEM_SHARED((_NUM_DOMAINS, _STYLE_DIM), jnp.float32),
            pltpu.SemaphoreType.DMA,
            pltpu.SemaphoreType.DMA,
            pltpu.SemaphoreType.DMA,
            pltpu.SemaphoreType.DMA,
        ],
    )
    def gather(dom_tab, sty_tab, dom_id, sty_id, dom_out, sty_out,
               idx_d, idx_s, dom_v, sty_v, dom_sh, sd0, sd1, ss0, ss1):
        sid = lax.axis_index("s")
        wid = sid * info.num_cores + lax.axis_index("c")
        base = wid * b_per_w
        sem_d = (sd0, sd1)
        sem_s = (ss0, ss1)

        @pl.when(sid == 0)
        def _():
            pltpu.sync_copy(dom_tab, dom_sh)

        pltpu.sync_copy(dom_id.at[wid], idx_d)
        pltpu.sync_copy(sty_id.at[wid], idx_s)
        plsc.subcore_barrier()

        def fire(j, buf):
            cd = pltpu.async_copy(dom_sh.at[idx_d.at[j]], dom_v.at[buf],
                                  sem_d[buf])
            cs = pltpu.async_copy(sty_tab.at[idx_s.at[j]], sty_v.at[buf],
                                  sem_s[buf])
            return cd, cs

        pend = fire(0, 0)
        for j in range(nch):
            cur = pend
            if j + 1 < nch:
                pend = fire(j + 1, (j + 1) % 2)
            cur[0].wait()
            cur[1].wait()
            buf = j % 2
            pltpu.sync_copy(dom_v.at[buf],
                            dom_out.at[pl.ds(base + j * _CHUNK, _CHUNK)])
            pltpu.sync_copy(sty_v.at[buf],
                            sty_out.at[pl.ds(base + j * _CHUNK, _CHUNK)])

    def run(dom_tab, sty_tab, dom_id, sty_id):
        dom_tab_p = jnp.pad(dom_tab, ((0, 0), (0, _STYLE_DIM - _DOM_DIM)))
        dom_id_r = dom_id.astype(jnp.int32).reshape(nw, nch, _CHUNK)
        sty_id_r = sty_id.astype(jnp.int32).reshape(nw, nch, _CHUNK)
        return gather(dom_tab_p, sty_tab, dom_id_r, sty_id_r)

    return run


def _mlp_body(dom_ref, sty_ref, w1a_ref, w1b_ref, b1_ref, w2_ref, b2_ref,
              out_ref):
    h = (jnp.dot(dom_ref[:, :_DOM_DIM], w1a_ref[...],
                 preferred_element_type=jnp.float32)
         + jnp.dot(sty_ref[...], w1b_ref[...],
                   preferred_element_type=jnp.float32)
         + b1_ref[...])
    a = h * 0.5 * (1.0 + jnp.tanh(0.5 * h))
    out_ref[...] = (jnp.dot(a, w2_ref[...],
                            preferred_element_type=jnp.float32)
                    + b2_ref[...])


def _mlp(dom, sty, w1a, w1b, b1, w2, b2, block_b=1024):
    batch = dom.shape[0]
    nblk = batch // block_b
    return pl.pallas_call(
        _mlp_body,
        grid=(nblk,),
        in_specs=[
            pl.BlockSpec((block_b, _STYLE_DIM), lambda i: (i, 0)),
            pl.BlockSpec((block_b, _STYLE_DIM), lambda i: (i, 0)),
            pl.BlockSpec((_DOM_DIM, _COND_DIM), lambda i: (0, 0)),
            pl.BlockSpec((_STYLE_DIM, _COND_DIM), lambda i: (0, 0)),
            pl.BlockSpec((1, _COND_DIM), lambda i: (0, 0)),
            pl.BlockSpec((_COND_DIM, _COND_DIM), lambda i: (0, 0)),
            pl.BlockSpec((1, _COND_DIM), lambda i: (0, 0)),
        ],
        out_specs=pl.BlockSpec((block_b, _COND_DIM), lambda i: (i, 0)),
        out_shape=jax.ShapeDtypeStruct((batch, _COND_DIM), jnp.float32),
        compiler_params=pltpu.CompilerParams(
            dimension_semantics=("arbitrary",),
        ),
    )(dom, sty, w1a, w1b, b1, w2, b2)


_gather = _make_gather(_BATCH)


def kernel(domain_id, style_id, domain_table, style_table, W1, b1, W2, b2):
    dom, sty = _gather(domain_table, style_table, domain_id, style_id)
    return _mlp(dom, sty, W1[:_DOM_DIM], W1[_DOM_DIM:], b1.reshape(1, -1),
                W2, b2.reshape(1, -1))
